# baseline (speedup 1.0000x reference)
import functools

import jax
import jax.numpy as jnp
from jax.experimental import pallas as pl
from jax.experimental.pallas import tpu as pltpu

DIM = 128
S = 16
B = 1024
RB = 256
NSTEPS = (B * S) // RB


def _body(src_ref, sneg_ref, snn_ref, dst_ref, dneg_ref, dnn_ref,
          w2_ref, d1_ref, d2_ref, d3_ref, d4_ref,
          out_ref, sacc, dacc):
    i = pl.program_id(0)
    w2a = w2_ref[:DIM, :]
    w2b = w2_ref[DIM:, :]

    inv_s = jnp.float32(1.0 / S)

    sm = snn_ref[...].reshape(RB, S, DIM).sum(axis=1) * inv_s
    sn2 = (jnp.dot(sneg_ref[...], w2a, preferred_element_type=jnp.float32)
           + jnp.dot(sm, w2b, preferred_element_type=jnp.float32))
    dm = dnn_ref[...].reshape(RB, S, DIM).sum(axis=1) * inv_s
    dn2 = (jnp.dot(dneg_ref[...], w2a, preferred_element_type=jnp.float32)
           + jnp.dot(dm, w2b, preferred_element_type=jnp.float32))

    rows = RB // S
    sacc[pl.ds(i * rows, rows), :] = sn2.reshape(rows, S, DIM).sum(axis=1) * inv_s
    dacc[pl.ds(i * rows, rows), :] = dn2.reshape(rows, S, DIM).sum(axis=1) * inv_s

    @pl.when(i == NSTEPS - 1)
    def _tail():
        src2 = (jnp.dot(src_ref[...], w2a, preferred_element_type=jnp.float32)
                + jnp.dot(sacc[...], w2b, preferred_element_type=jnp.float32))
        dst2 = (jnp.dot(dst_ref[...], w2a, preferred_element_type=jnp.float32)
                + jnp.dot(dacc[...], w2b, preferred_element_type=jnp.float32))
        h = jax.nn.relu(
            jnp.dot(src2, d1_ref[:DIM, :], preferred_element_type=jnp.float32)
            + jnp.dot(dst2, d1_ref[DIM:, :], preferred_element_type=jnp.float32))
        h = jax.nn.relu(jnp.dot(h, d2_ref[...], preferred_element_type=jnp.float32))
        h = jax.nn.relu(jnp.dot(h, d3_ref[...], preferred_element_type=jnp.float32))
        logits = jnp.dot(h, d4_ref[...], preferred_element_type=jnp.float32)
        m = jnp.max(logits, axis=-1, keepdims=True)
        e = jnp.exp(logits - m)
        out_ref[...] = e / jnp.sum(e, axis=-1, keepdims=True)


@jax.jit
def kernel(src, src_neg, src_neg_neg, dst, dst_neg, dst_neg_neg,
           w2, d1, d2, d3, d4):
    resident = lambda shape: pl.BlockSpec(shape, lambda i: (0, 0))
    return pl.pallas_call(
        _body,
        grid=(NSTEPS,),
        in_specs=[
            resident((B, DIM)),
            pl.BlockSpec((RB, DIM), lambda i: (i, 0)),
            pl.BlockSpec((RB * S, DIM), lambda i: (i, 0)),
            resident((B, DIM)),
            pl.BlockSpec((RB, DIM), lambda i: (i, 0)),
            pl.BlockSpec((RB * S, DIM), lambda i: (i, 0)),
            resident((2 * DIM, DIM)),
            resident((2 * DIM, 128)),
            resident((128, 64)),
            resident((64, 8)),
            resident((8, 2)),
        ],
        out_specs=pl.BlockSpec((B, 2), lambda i: (0, 0)),
        out_shape=jax.ShapeDtypeStruct((B, 2), jnp.float32),
        scratch_shapes=[
            pltpu.VMEM((B, DIM), jnp.float32),
            pltpu.VMEM((B, DIM), jnp.float32),
        ],
        compiler_params=pltpu.CompilerParams(
            dimension_semantics=("arbitrary",),
        ),
    )(src, src_neg, src_neg_neg, dst, dst_neg, dst_neg_neg,
      w2, d1, d2, d3, d4)


# speedup vs baseline: 1.1848x; 1.1848x over previous
import functools

import jax
import jax.numpy as jnp
from jax.experimental import pallas as pl
from jax.experimental.pallas import tpu as pltpu

DIM = 128
S = 16
B = 1024
RB = 1024
NSTEPS = (B * S) // RB


def _body(src_ref, sneg_ref, snn_ref, dst_ref, dneg_ref, dnn_ref,
          w2_ref, d1_ref, d2_ref, d3_ref, d4_ref,
          out_ref, sacc, dacc):
    i = pl.program_id(0)
    w2a = w2_ref[:DIM, :]
    w2b = w2_ref[DIM:, :]

    inv_s = jnp.float32(1.0 / S)

    sm = snn_ref[...].reshape(RB, S, DIM).sum(axis=1) * inv_s
    sn2 = (jnp.dot(sneg_ref[...], w2a, preferred_element_type=jnp.float32)
           + jnp.dot(sm, w2b, preferred_element_type=jnp.float32))
    dm = dnn_ref[...].reshape(RB, S, DIM).sum(axis=1) * inv_s
    dn2 = (jnp.dot(dneg_ref[...], w2a, preferred_element_type=jnp.float32)
           + jnp.dot(dm, w2b, preferred_element_type=jnp.float32))

    rows = RB // S
    sacc[pl.ds(i * rows, rows), :] = sn2.reshape(rows, S, DIM).sum(axis=1) * inv_s
    dacc[pl.ds(i * rows, rows), :] = dn2.reshape(rows, S, DIM).sum(axis=1) * inv_s

    @pl.when(i == NSTEPS - 1)
    def _tail():
        src2 = (jnp.dot(src_ref[...], w2a, preferred_element_type=jnp.float32)
                + jnp.dot(sacc[...], w2b, preferred_element_type=jnp.float32))
        dst2 = (jnp.dot(dst_ref[...], w2a, preferred_element_type=jnp.float32)
                + jnp.dot(dacc[...], w2b, preferred_element_type=jnp.float32))
        h = jax.nn.relu(
            jnp.dot(src2, d1_ref[:DIM, :], preferred_element_type=jnp.float32)
            + jnp.dot(dst2, d1_ref[DIM:, :], preferred_element_type=jnp.float32))
        h = jax.nn.relu(jnp.dot(h, d2_ref[...], preferred_element_type=jnp.float32))
        h = jax.nn.relu(jnp.dot(h, d3_ref[...], preferred_element_type=jnp.float32))
        logits = jnp.dot(h, d4_ref[...], preferred_element_type=jnp.float32)
        m = jnp.max(logits, axis=-1, keepdims=True)
        e = jnp.exp(logits - m)
        out_ref[...] = e / jnp.sum(e, axis=-1, keepdims=True)


@jax.jit
def kernel(src, src_neg, src_neg_neg, dst, dst_neg, dst_neg_neg,
           w2, d1, d2, d3, d4):
    resident = lambda shape: pl.BlockSpec(shape, lambda i: (0, 0))
    return pl.pallas_call(
        _body,
        grid=(NSTEPS,),
        in_specs=[
            resident((B, DIM)),
            pl.BlockSpec((RB, DIM), lambda i: (i, 0)),
            pl.BlockSpec((RB * S, DIM), lambda i: (i, 0)),
            resident((B, DIM)),
            pl.BlockSpec((RB, DIM), lambda i: (i, 0)),
            pl.BlockSpec((RB * S, DIM), lambda i: (i, 0)),
            resident((2 * DIM, DIM)),
            resident((2 * DIM, 128)),
            resident((128, 64)),
            resident((64, 8)),
            resident((8, 2)),
        ],
        out_specs=pl.BlockSpec((B, 2), lambda i: (0, 0)),
        out_shape=jax.ShapeDtypeStruct((B, 2), jnp.float32),
        scratch_shapes=[
            pltpu.VMEM((B, DIM), jnp.float32),
            pltpu.VMEM((B, DIM), jnp.float32),
        ],
        compiler_params=pltpu.CompilerParams(
            dimension_semantics=("arbitrary",),
        ),
    )(src, src_neg, src_neg_neg, dst, dst_neg, dst_neg_neg,
      w2, d1, d2, d3, d4)
